# baseline (speedup 1.0000x reference)
import functools

import jax
import jax.numpy as jnp
from jax import lax
from jax.experimental import pallas as pl
from jax.experimental.pallas import tpu as pltpu
from jax.experimental.pallas import tpu_sc as plsc

NC = 2
NS = 16
L = 16
NW = NC * NS


def _proj_body(x_ref, w_ref, as_ref, ad_ref, xw0_ref, xw1_ref, s_ref, d_ref):
    xw = jnp.dot(x_ref[...], w_ref[...], preferred_element_type=jnp.float32)
    h = xw.shape[1] // 2
    xw0_ref[...] = xw[:, :h]
    xw1_ref[...] = xw[:, h:]
    s_ref[...] = jnp.dot(xw, as_ref[...], preferred_element_type=jnp.float32)
    d_ref[...] = jnp.dot(xw, ad_ref[...], preferred_element_type=jnp.float32)


def _project(x_pad, W, att_src, att_dst):
    NP, D = x_pad.shape
    H = D // 2
    RB = NP // 8
    return pl.pallas_call(
        _proj_body,
        grid=(NP // RB,),
        in_specs=[
            pl.BlockSpec((RB, D), lambda i: (i, 0)),
            pl.BlockSpec((D, D), lambda i: (0, 0)),
            pl.BlockSpec((D, 1), lambda i: (0, 0)),
            pl.BlockSpec((D, 1), lambda i: (0, 0)),
        ],
        out_specs=[
            pl.BlockSpec((RB, H), lambda i: (i, 0)),
            pl.BlockSpec((RB, H), lambda i: (i, 0)),
            pl.BlockSpec((RB, 1), lambda i: (i, 0)),
            pl.BlockSpec((RB, 1), lambda i: (i, 0)),
        ],
        out_shape=[
            jax.ShapeDtypeStruct((NP, H), jnp.float32),
            jax.ShapeDtypeStruct((NP, H), jnp.float32),
            jax.ShapeDtypeStruct((NP, 1), jnp.float32),
            jax.ShapeDtypeStruct((NP, 1), jnp.float32),
        ],
    )(x_pad, W, att_src.reshape(D, 1), att_dst.reshape(D, 1))


def _edge_ex(asrc, adst, src2, dst2, NP, EW, NB):
    RT = NP // NS

    mesh = plsc.VectorSubcoreMesh(core_axis_name="c", subcore_axis_name="s")

    @functools.partial(
        pl.kernel,
        mesh=mesh,
        out_type=[
            jax.ShapeDtypeStruct((NC * NP,), jnp.float32),
            jax.ShapeDtypeStruct((NW, 1, EW), jnp.float32),
        ],
        scratch_types=[
            pltpu.VMEM((NP,), jnp.float32),
            pltpu.VMEM((NP,), jnp.float32),
            pltpu.VMEM((NB, 128), jnp.int32),
            pltpu.VMEM((NB, 128), jnp.int32),
            pltpu.VMEM((EW,), jnp.float32),
            pltpu.VMEM_SHARED((NP,), jnp.float32),
        ],
        compiler_params=pltpu.CompilerParams(needs_layout_passes=False),
    )
    def k(asrc_hbm, adst_hbm, src2_hbm, dst2_hbm,
          denomp_hbm, ex_hbm,
          asrc_v, adst_v, src2_v, dst2_v, exf_v, denom_sh):
        cid = lax.axis_index("c")
        sid = lax.axis_index("s")
        wid = sid * NC + cid

        pltpu.sync_copy(asrc_hbm, asrc_v)
        pltpu.sync_copy(adst_hbm, adst_v)
        pltpu.sync_copy(src2_hbm.at[wid], src2_v)
        pltpu.sync_copy(dst2_hbm.at[wid], dst2_v)
        zv = jnp.zeros((L,), jnp.float32)

        def zbody(i, carry):
            exf_v[pl.ds(i * L, L)] = zv
            return carry

        lax.fori_loop(0, (RT + L - 1) // L, zbody, 0)
        pltpu.sync_copy(exf_v.at[pl.ds(0, RT)],
                        denom_sh.at[pl.ds(sid * RT, RT)])
        plsc.subcore_barrier()

        def maxs_body(i, m):
            return jnp.maximum(m, asrc_v[pl.ds(i * L, L)])

        def maxd_body(i, m):
            return jnp.maximum(m, adst_v[pl.ds(i * L, L)])

        ms = lax.fori_loop(1, NP // L, maxs_body, asrc_v[pl.ds(0, L)])
        md = lax.fori_loop(1, NP // L, maxd_body, adst_v[pl.ds(0, L)])
        smax = jnp.max(ms) + jnp.max(md)
        cs = jnp.where(smax >= 0.0, smax, smax * 0.2)
        cv = jnp.full((L,), cs, jnp.float32)

        def row_body(j, carry):
            for kk in range(128 // L):
                off = j * 128 + kk * L
                sv = src2_v[j, pl.ds(kk * L, L)]
                dv = dst2_v[j, pl.ds(kk * L, L)]
                s = (plsc.load_gather(asrc_v, [sv])
                     + plsc.load_gather(adst_v, [dv]))
                alpha = jnp.where(s >= 0.0, s, s * 0.2)
                exf_v[pl.ds(off, L)] = jnp.exp(alpha - cv)
            pltpu.sync_copy(exf_v.at[pl.ds(j * 128, 128)],
                            denom_sh.at[dst2_v.at[j]], add=True)
            return carry

        lax.fori_loop(0, NB, row_body, 0)
        pltpu.sync_copy(exf_v, ex_hbm.at[wid, 0])
        plsc.subcore_barrier()
        pltpu.sync_copy(denom_sh.at[pl.ds(sid * RT, RT)],
                        exf_v.at[pl.ds(0, RT)])
        pltpu.sync_copy(exf_v.at[pl.ds(0, RT)],
                        denomp_hbm.at[pl.ds(cid * NP + sid * RT, RT)])

    return k(asrc, adst, src2, dst2)


def _edge_msg(xw0, xw1, src2, dst2, ex, denomp, NP, EW2, NB2, D):
    RT = NP // NS
    H = D // 2

    mesh = plsc.VectorSubcoreMesh(core_axis_name="c", subcore_axis_name="s")

    @functools.partial(
        pl.kernel,
        mesh=mesh,
        out_type=jax.ShapeDtypeStruct((NC, NP, H), jnp.float32),
        scratch_types=[
            pltpu.VMEM((NP,), jnp.float32),
            pltpu.VMEM((EW2,), jnp.float32),
            pltpu.VMEM((3, 128), jnp.int32),
            pltpu.VMEM((NB2, 128), jnp.int32),
            pltpu.VMEM((128, H), jnp.float32),
            pltpu.VMEM((128, H), jnp.float32),
            pltpu.VMEM((128, H), jnp.float32),
            pltpu.SemaphoreType.DMA,
            pltpu.SemaphoreType.DMA,
            pltpu.SemaphoreType.DMA,
            pltpu.SemaphoreType.DMA,
            pltpu.SemaphoreType.DMA,
            pltpu.SemaphoreType.DMA,
            pltpu.VMEM_SHARED((NP, H), jnp.float32),
        ],
        compiler_params=pltpu.CompilerParams(needs_layout_passes=False,
                                             use_tc_tiling_on_sc=False),
    )
    def k(xw0_hbm, xw1_hbm, src2_hbm, dst2_hbm, ex_hbm, denomp_hbm,
          outp_hbm,
          den0_v, coef_v, srcring_v, dst2_v, rows_v, rows2_v, rows3_v,
          sem, sem2, sem3, isem, isem2, isem3, out_sh):
        cid = lax.axis_index("c")
        sid = lax.axis_index("s")

        pltpu.sync_copy(denomp_hbm.at[pl.ds(0, NP)], den0_v)
        pltpu.sync_copy(denomp_hbm.at[pl.ds(NP, NP)], coef_v.at[pl.ds(0, NP)])
        pltpu.sync_copy(dst2_hbm.at[sid], dst2_v)
        zv = jnp.zeros((L,), jnp.float32)

        def zbody(r, carry):
            for kk in range(H // L):
                rows_v[r, pl.ds(kk * L, L)] = zv
            return carry

        lax.fori_loop(0, 128, zbody, 0)
        nfull = RT // 128
        rem = RT % 128
        for q in range(nfull):
            pltpu.sync_copy(rows_v,
                            out_sh.at[pl.ds(sid * RT + q * 128, 128)])
        if rem:
            pltpu.sync_copy(rows_v.at[pl.ds(0, rem)],
                            out_sh.at[pl.ds(sid * RT + nfull * 128, rem)])

        def dbody(i, carry):
            sl = pl.ds(i * L, L)
            den0_v[sl] = 1.0 / (den0_v[sl] + coef_v[sl] + 1e-16)
            return carry

        lax.fori_loop(0, NP // L, dbody, 0)
        pltpu.sync_copy(ex_hbm.at[sid, 0], coef_v)

        def cbody(j, carry):
            for kk in range(128 // L):
                off = j * 128 + kk * L
                dv = dst2_v[j, pl.ds(kk * L, L)]
                r = plsc.load_gather(den0_v, [dv])
                coef_v[pl.ds(off, L)] = coef_v[pl.ds(off, L)] * r
            return carry

        lax.fori_loop(0, NB2, cbody, 0)
        plsc.subcore_barrier()

        def scale(buf, j):
            def gbody(g, c2):
                base = j * 128 + g * L
                for r16 in range(L):
                    r = g * L + r16
                    cb = plsc.load_gather(
                        coef_v, [jnp.full((L,), base + r16, jnp.int32)])
                    for kk in range(H // L):
                        sl = pl.ds(kk * L, L)
                        buf[r, sl] = buf[r, sl] * cb
                return c2

            lax.fori_loop(0, 128 // L, gbody, 0)

        bufs = (rows_v, rows2_v, rows3_v)
        sems = (sem, sem2, sem3)
        isems = (isem, isem2, isem3)

        def run(xwh_hbm):
            for b in range(min(3, NB2)):
                pltpu.async_copy(src2_hbm.at[sid, b], srcring_v.at[b],
                                 isems[b])
            for b in range(min(2, NB2)):
                pltpu.make_async_copy(src2_hbm.at[sid, b], srcring_v.at[b],
                                      isems[b]).wait()
                pltpu.async_copy(xwh_hbm.at[srcring_v.at[b]], bufs[b],
                                 sems[b])

            def pbody(t, carry):
                for ph in range(3):
                    j = 3 * t + ph
                    pltpu.make_async_copy(
                        xwh_hbm.at[srcring_v.at[ph]], bufs[ph],
                        sems[ph]).wait()

                    @pl.when(j + 3 < NB2)
                    def _(ph=ph, j=j):
                        pltpu.async_copy(src2_hbm.at[sid, j + 3],
                                         srcring_v.at[ph], isems[ph])

                    scale(bufs[ph], j)
                    pltpu.sync_copy(bufs[ph], out_sh.at[dst2_v.at[j]],
                                    add=True)

                    pn = (ph + 2) % 3

                    @pl.when(j + 2 < NB2)
                    def _(ph=ph, j=j, pn=pn):
                        pltpu.make_async_copy(
                            src2_hbm.at[sid, j + 2], srcring_v.at[pn],
                            isems[pn]).wait()
                        pltpu.async_copy(
                            xwh_hbm.at[srcring_v.at[pn]], bufs[pn],
                            sems[pn])
                return carry

            lax.fori_loop(0, NB2 // 3, pbody, 0)
            base = (NB2 // 3) * 3
            for q in range(NB2 % 3):
                j = base + q
                b = j % 3
                pltpu.make_async_copy(
                    xwh_hbm.at[srcring_v.at[b]], bufs[b], sems[b]).wait()
                scale(bufs[b], j)
                pltpu.sync_copy(bufs[b], out_sh.at[dst2_v.at[j]], add=True)

        @pl.when(cid == 0)
        def _():
            run(xw0_hbm)

        @pl.when(cid == 1)
        def _():
            run(xw1_hbm)

        plsc.subcore_barrier()
        for q in range(nfull):
            base = sid * RT + q * 128
            pltpu.sync_copy(out_sh.at[pl.ds(base, 128)], rows_v)
            pltpu.sync_copy(rows_v, outp_hbm.at[cid, pl.ds(base, 128)])
        if rem:
            base = sid * RT + nfull * 128
            pltpu.sync_copy(out_sh.at[pl.ds(base, rem)],
                            rows_v.at[pl.ds(0, rem)])
            pltpu.sync_copy(rows_v.at[pl.ds(0, rem)],
                            outp_hbm.at[cid, pl.ds(base, rem)])

    return k(xw0, xw1, src2, dst2, ex, denomp)


def _fin_body(p_ref, x_ref, b_ref, o_ref):
    p = p_ref[...]
    h = jnp.concatenate([p[0], p[1]], axis=1) + b_ref[...]
    o_ref[...] = x_ref[...] + jnp.where(h > 0.0, h, jnp.exp(h) - 1.0)


def _finalize(outp, x, bias, R):
    N, D = x.shape
    H = D // 2
    return pl.pallas_call(
        _fin_body,
        grid=(N // R,),
        in_specs=[
            pl.BlockSpec((2, R, H), lambda i: (0, i, 0)),
            pl.BlockSpec((R, D), lambda i: (i, 0)),
            pl.BlockSpec((1, D), lambda i: (0, 0)),
        ],
        out_specs=pl.BlockSpec((R, D), lambda i: (i, 0)),
        out_shape=jax.ShapeDtypeStruct((N, D), jnp.float32),
    )(outp, x, bias.reshape(1, D))


def kernel(x, edge_index, W, att_src, att_dst, bias):
    N, D = x.shape
    E = edge_index.shape[1]
    NP = ((N + 1 + 127) // 128) * 128
    ET0 = E + N
    NB = -(-ET0 // (NW * 128))
    EW = NB * 128
    ET = NW * EW

    loop = jnp.arange(N, dtype=jnp.int32)
    pad = ET - ET0
    src_all = jnp.concatenate(
        [edge_index[0], loop, jnp.zeros((pad,), jnp.int32)])
    dst_all = jnp.concatenate(
        [edge_index[1], loop, jnp.full((pad,), N, jnp.int32)])
    src2 = src_all.reshape(NW, NB, 128)
    dst2 = dst_all.reshape(NW, NB, 128)

    x_pad = jnp.zeros((NP, D), jnp.float32).at[:N].set(x)
    xw0, xw1, as_col, ad_col = _project(x_pad, W, att_src, att_dst)
    asrc = as_col.reshape(NP)
    adst = ad_col.reshape(NP)

    denomp, ex = _edge_ex(asrc, adst, src2, dst2, NP, EW, NB)

    EW2 = NC * EW
    NB2 = NC * NB
    srcb2 = src_all.reshape(NS, NB2, 128)
    dstb2 = dst_all.reshape(NS, NB2, 128)
    exb = ex.reshape(NS, 1, EW2)
    outp = _edge_msg(xw0, xw1, srcb2, dstb2, exb, denomp, NP, EW2, NB2, D)

    R = next((r for r in range(512, 7, -8) if N % r == 0), None)
    if R is None:
        R = next(r for r in (7, 5, 4, 2, 1) if N % r == 0)
    return _finalize(outp, x, bias, R)


# speedup vs baseline: 1.1318x; 1.1318x over previous
import functools

import jax
import jax.numpy as jnp
from jax import lax
from jax.experimental import pallas as pl
from jax.experimental.pallas import tpu as pltpu
from jax.experimental.pallas import tpu_sc as plsc

NC = 2
NS = 16
L = 16
NW = NC * NS


def _proj_body(x_ref, w_ref, as_ref, ad_ref, xw0_ref, xw1_ref, s_ref, d_ref):
    xw = jnp.dot(x_ref[...], w_ref[...], preferred_element_type=jnp.float32)
    h = xw.shape[1] // 2
    xw0_ref[...] = xw[:, :h]
    xw1_ref[...] = xw[:, h:]
    s_ref[...] = jnp.dot(xw, as_ref[...], preferred_element_type=jnp.float32)
    d_ref[...] = jnp.dot(xw, ad_ref[...], preferred_element_type=jnp.float32)


def _project(x_pad, W, att_src, att_dst):
    NP, D = x_pad.shape
    H = D // 2
    RB = NP // 8
    return pl.pallas_call(
        _proj_body,
        grid=(NP // RB,),
        in_specs=[
            pl.BlockSpec((RB, D), lambda i: (i, 0)),
            pl.BlockSpec((D, D), lambda i: (0, 0)),
            pl.BlockSpec((D, 1), lambda i: (0, 0)),
            pl.BlockSpec((D, 1), lambda i: (0, 0)),
        ],
        out_specs=[
            pl.BlockSpec((RB, H), lambda i: (i, 0)),
            pl.BlockSpec((RB, H), lambda i: (i, 0)),
            pl.BlockSpec((RB, 1), lambda i: (i, 0)),
            pl.BlockSpec((RB, 1), lambda i: (i, 0)),
        ],
        out_shape=[
            jax.ShapeDtypeStruct((NP, H), jnp.float32),
            jax.ShapeDtypeStruct((NP, H), jnp.float32),
            jax.ShapeDtypeStruct((NP, 1), jnp.float32),
            jax.ShapeDtypeStruct((NP, 1), jnp.float32),
        ],
    )(x_pad, W, att_src.reshape(D, 1), att_dst.reshape(D, 1))


def _edge_ex(asrc, adst, src2, dst2, NP, EW, NB):
    RT = NP // NS

    mesh = plsc.VectorSubcoreMesh(core_axis_name="c", subcore_axis_name="s")

    @functools.partial(
        pl.kernel,
        mesh=mesh,
        out_type=[
            jax.ShapeDtypeStruct((NC * NP,), jnp.float32),
            jax.ShapeDtypeStruct((NW, 1, EW), jnp.float32),
        ],
        scratch_types=[
            pltpu.VMEM((NP,), jnp.float32),
            pltpu.VMEM((NP,), jnp.float32),
            pltpu.VMEM((NB, 128), jnp.int32),
            pltpu.VMEM((NB, 128), jnp.int32),
            pltpu.VMEM((EW,), jnp.float32),
            pltpu.VMEM_SHARED((NP,), jnp.float32),
        ],
        compiler_params=pltpu.CompilerParams(needs_layout_passes=False),
    )
    def k(asrc_hbm, adst_hbm, src2_hbm, dst2_hbm,
          denomp_hbm, ex_hbm,
          asrc_v, adst_v, src2_v, dst2_v, exf_v, denom_sh):
        cid = lax.axis_index("c")
        sid = lax.axis_index("s")
        wid = sid * NC + cid

        pltpu.sync_copy(asrc_hbm, asrc_v)
        pltpu.sync_copy(adst_hbm, adst_v)
        pltpu.sync_copy(src2_hbm.at[wid], src2_v)
        pltpu.sync_copy(dst2_hbm.at[wid], dst2_v)
        zv = jnp.zeros((L,), jnp.float32)

        def zbody(i, carry):
            exf_v[pl.ds(i * L, L)] = zv
            return carry

        lax.fori_loop(0, (RT + L - 1) // L, zbody, 0)
        pltpu.sync_copy(exf_v.at[pl.ds(0, RT)],
                        denom_sh.at[pl.ds(sid * RT, RT)])
        plsc.subcore_barrier()

        def maxs_body(i, m):
            return jnp.maximum(m, asrc_v[pl.ds(i * L, L)])

        def maxd_body(i, m):
            return jnp.maximum(m, adst_v[pl.ds(i * L, L)])

        ms = lax.fori_loop(1, NP // L, maxs_body, asrc_v[pl.ds(0, L)])
        md = lax.fori_loop(1, NP // L, maxd_body, adst_v[pl.ds(0, L)])
        smax = jnp.max(ms) + jnp.max(md)
        cs = jnp.where(smax >= 0.0, smax, smax * 0.2)
        cv = jnp.full((L,), cs, jnp.float32)

        def row_body(j, carry):
            for kk in range(128 // L):
                off = j * 128 + kk * L
                sv = src2_v[j, pl.ds(kk * L, L)]
                dv = dst2_v[j, pl.ds(kk * L, L)]
                s = (plsc.load_gather(asrc_v, [sv])
                     + plsc.load_gather(adst_v, [dv]))
                alpha = jnp.where(s >= 0.0, s, s * 0.2)
                exf_v[pl.ds(off, L)] = jnp.exp(alpha - cv)
            pltpu.sync_copy(exf_v.at[pl.ds(j * 128, 128)],
                            denom_sh.at[dst2_v.at[j]], add=True)
            return carry

        lax.fori_loop(0, NB, row_body, 0)
        pltpu.sync_copy(exf_v, ex_hbm.at[wid, 0])
        plsc.subcore_barrier()
        pltpu.sync_copy(denom_sh.at[pl.ds(sid * RT, RT)],
                        exf_v.at[pl.ds(0, RT)])
        pltpu.sync_copy(exf_v.at[pl.ds(0, RT)],
                        denomp_hbm.at[pl.ds(cid * NP + sid * RT, RT)])

    return k(asrc, adst, src2, dst2)


def _edge_msg(xw0, xw1, src2, dst2, ex, denomp, NP, EW2, NB2, D):
    RT = NP // NS
    H = D // 2

    mesh = plsc.VectorSubcoreMesh(core_axis_name="c", subcore_axis_name="s")

    @functools.partial(
        pl.kernel,
        mesh=mesh,
        out_type=jax.ShapeDtypeStruct((NC, NP, H), jnp.float32),
        scratch_types=[
            pltpu.VMEM((NP,), jnp.float32),
            pltpu.VMEM((EW2,), jnp.float32),
            pltpu.VMEM((3, 128), jnp.int32),
            pltpu.VMEM((NB2, 128), jnp.int32),
            pltpu.VMEM((128, H), jnp.float32),
            pltpu.VMEM((128, H), jnp.float32),
            pltpu.VMEM((128, H), jnp.float32),
            pltpu.SemaphoreType.DMA,
            pltpu.SemaphoreType.DMA,
            pltpu.SemaphoreType.DMA,
            pltpu.SemaphoreType.DMA,
            pltpu.SemaphoreType.DMA,
            pltpu.SemaphoreType.DMA,
            pltpu.SemaphoreType.DMA,
            pltpu.SemaphoreType.DMA,
            pltpu.SemaphoreType.DMA,
            pltpu.VMEM_SHARED((NP, H), jnp.float32),
        ],
        compiler_params=pltpu.CompilerParams(needs_layout_passes=False,
                                             use_tc_tiling_on_sc=False),
    )
    def k(xw0_hbm, xw1_hbm, src2_hbm, dst2_hbm, ex_hbm, denomp_hbm,
          outp_hbm,
          den0_v, coef_v, srcring_v, dst2_v, rows_v, rows2_v, rows3_v,
          sem, sem2, sem3, isem, isem2, isem3, ssem, ssem2, ssem3, out_sh):
        cid = lax.axis_index("c")
        sid = lax.axis_index("s")

        pltpu.sync_copy(denomp_hbm.at[pl.ds(0, NP)], den0_v)
        pltpu.sync_copy(denomp_hbm.at[pl.ds(NP, NP)], coef_v.at[pl.ds(0, NP)])
        pltpu.sync_copy(dst2_hbm.at[sid], dst2_v)
        zv = jnp.zeros((L,), jnp.float32)

        def zbody(r, carry):
            for kk in range(H // L):
                rows_v[r, pl.ds(kk * L, L)] = zv
            return carry

        lax.fori_loop(0, 128, zbody, 0)
        nfull = RT // 128
        rem = RT % 128
        for q in range(nfull):
            pltpu.sync_copy(rows_v,
                            out_sh.at[pl.ds(sid * RT + q * 128, 128)])
        if rem:
            pltpu.sync_copy(rows_v.at[pl.ds(0, rem)],
                            out_sh.at[pl.ds(sid * RT + nfull * 128, rem)])

        def dbody(i, carry):
            sl = pl.ds(i * L, L)
            den0_v[sl] = 1.0 / (den0_v[sl] + coef_v[sl] + 1e-16)
            return carry

        lax.fori_loop(0, NP // L, dbody, 0)
        pltpu.sync_copy(ex_hbm.at[sid, 0], coef_v)

        def cbody(j, carry):
            for kk in range(128 // L):
                off = j * 128 + kk * L
                dv = dst2_v[j, pl.ds(kk * L, L)]
                r = plsc.load_gather(den0_v, [dv])
                coef_v[pl.ds(off, L)] = coef_v[pl.ds(off, L)] * r
            return carry

        lax.fori_loop(0, NB2, cbody, 0)
        plsc.subcore_barrier()

        def scale(buf, j):
            def gbody(g, c2):
                base = j * 128 + g * L
                for r16 in range(L):
                    r = g * L + r16
                    cb = plsc.load_gather(
                        coef_v, [jnp.full((L,), base + r16, jnp.int32)])
                    for kk in range(H // L):
                        sl = pl.ds(kk * L, L)
                        buf[r, sl] = buf[r, sl] * cb
                return c2

            lax.fori_loop(0, 128 // L, gbody, 0)

        bufs = (rows_v, rows2_v, rows3_v)
        sems = (sem, sem2, sem3)
        isems = (isem, isem2, isem3)
        ssems = (ssem, ssem2, ssem3)

        def run(xwh_hbm):
            for b in range(min(3, NB2)):
                pltpu.async_copy(src2_hbm.at[sid, b], srcring_v.at[b],
                                 isems[b])
            for b in range(min(2, NB2)):
                pltpu.make_async_copy(src2_hbm.at[sid, b], srcring_v.at[b],
                                      isems[b]).wait()
                pltpu.async_copy(xwh_hbm.at[srcring_v.at[b]], bufs[b],
                                 sems[b])

            def pbody(t, carry):
                for ph in range(3):
                    j = 3 * t + ph
                    pltpu.make_async_copy(
                        xwh_hbm.at[srcring_v.at[ph]], bufs[ph],
                        sems[ph]).wait()

                    @pl.when(j + 3 < NB2)
                    def _(ph=ph, j=j):
                        pltpu.async_copy(src2_hbm.at[sid, j + 3],
                                         srcring_v.at[ph], isems[ph])

                    scale(bufs[ph], j)
                    pltpu.async_copy(bufs[ph], out_sh.at[dst2_v.at[j]],
                                     ssems[ph], add=True)

                    pn = (ph + 2) % 3

                    @pl.when((j + 2 < NB2) & (j >= 1))
                    def _(j=j, pn=pn):
                        pltpu.make_async_copy(
                            bufs[pn], out_sh.at[dst2_v.at[j - 1]],
                            ssems[pn]).wait()

                    @pl.when(j + 2 < NB2)
                    def _(ph=ph, j=j, pn=pn):
                        pltpu.make_async_copy(
                            src2_hbm.at[sid, j + 2], srcring_v.at[pn],
                            isems[pn]).wait()
                        pltpu.async_copy(
                            xwh_hbm.at[srcring_v.at[pn]], bufs[pn],
                            sems[pn])
                return carry

            lax.fori_loop(0, NB2 // 3, pbody, 0)
            nw = 3 * (NB2 // 3)
            for j in range(max(0, nw - 3), nw):
                if not (j + 3 < NB2 and j + 1 >= 1):
                    b = j % 3
                    pltpu.make_async_copy(
                        bufs[b], out_sh.at[dst2_v.at[j]], ssems[b]).wait()
            base = (NB2 // 3) * 3
            for q in range(NB2 % 3):
                j = base + q
                b = j % 3
                pltpu.make_async_copy(
                    xwh_hbm.at[srcring_v.at[b]], bufs[b], sems[b]).wait()
                scale(bufs[b], j)
                pltpu.sync_copy(bufs[b], out_sh.at[dst2_v.at[j]], add=True)

        @pl.when(cid == 0)
        def _():
            run(xw0_hbm)

        @pl.when(cid == 1)
        def _():
            run(xw1_hbm)

        plsc.subcore_barrier()
        for q in range(nfull):
            base = sid * RT + q * 128
            pltpu.sync_copy(out_sh.at[pl.ds(base, 128)], rows_v)
            pltpu.sync_copy(rows_v, outp_hbm.at[cid, pl.ds(base, 128)])
        if rem:
            base = sid * RT + nfull * 128
            pltpu.sync_copy(out_sh.at[pl.ds(base, rem)],
                            rows_v.at[pl.ds(0, rem)])
            pltpu.sync_copy(rows_v.at[pl.ds(0, rem)],
                            outp_hbm.at[cid, pl.ds(base, rem)])

    return k(xw0, xw1, src2, dst2, ex, denomp)


def _fin_body(p_ref, x_ref, b_ref, o_ref):
    p = p_ref[...]
    h = jnp.concatenate([p[0], p[1]], axis=1) + b_ref[...]
    o_ref[...] = x_ref[...] + jnp.where(h > 0.0, h, jnp.exp(h) - 1.0)


def _finalize(outp, x, bias, R):
    N, D = x.shape
    H = D // 2
    return pl.pallas_call(
        _fin_body,
        grid=(N // R,),
        in_specs=[
            pl.BlockSpec((2, R, H), lambda i: (0, i, 0)),
            pl.BlockSpec((R, D), lambda i: (i, 0)),
            pl.BlockSpec((1, D), lambda i: (0, 0)),
        ],
        out_specs=pl.BlockSpec((R, D), lambda i: (i, 0)),
        out_shape=jax.ShapeDtypeStruct((N, D), jnp.float32),
    )(outp, x, bias.reshape(1, D))


def kernel(x, edge_index, W, att_src, att_dst, bias):
    N, D = x.shape
    E = edge_index.shape[1]
    NP = ((N + 1 + 127) // 128) * 128
    ET0 = E + N
    NB = -(-ET0 // (NW * 128))
    EW = NB * 128
    ET = NW * EW

    loop = jnp.arange(N, dtype=jnp.int32)
    pad = ET - ET0
    src_all = jnp.concatenate(
        [edge_index[0], loop, jnp.zeros((pad,), jnp.int32)])
    dst_all = jnp.concatenate(
        [edge_index[1], loop, jnp.full((pad,), N, jnp.int32)])
    src2 = src_all.reshape(NW, NB, 128)
    dst2 = dst_all.reshape(NW, NB, 128)

    x_pad = jnp.zeros((NP, D), jnp.float32).at[:N].set(x)
    xw0, xw1, as_col, ad_col = _project(x_pad, W, att_src, att_dst)
    asrc = as_col.reshape(NP)
    adst = ad_col.reshape(NP)

    denomp, ex = _edge_ex(asrc, adst, src2, dst2, NP, EW, NB)

    EW2 = NC * EW
    NB2 = NC * NB
    srcb2 = src_all.reshape(NS, NB2, 128)
    dstb2 = dst_all.reshape(NS, NB2, 128)
    exb = ex.reshape(NS, 1, EW2)
    outp = _edge_msg(xw0, xw1, srcb2, dstb2, exb, denomp, NP, EW2, NB2, D)

    R = next((r for r in range(512, 7, -8) if N % r == 0), None)
    if R is None:
        R = next(r for r in (7, 5, 4, 2, 1) if N % r == 0)
    return _finalize(outp, x, bias, R)


# speedup vs baseline: 1.1881x; 1.0498x over previous
import functools

import jax
import jax.numpy as jnp
from jax import lax
from jax.experimental import pallas as pl
from jax.experimental.pallas import tpu as pltpu
from jax.experimental.pallas import tpu_sc as plsc

NC = 2
NS = 16
L = 16
NW = NC * NS


def _proj_body(x_ref, w_ref, as_ref, ad_ref, xw0_ref, xw1_ref, s_ref, d_ref):
    xw = jnp.dot(x_ref[...], w_ref[...], preferred_element_type=jnp.float32)
    h = xw.shape[1] // 2
    xw0_ref[...] = xw[:, :h]
    xw1_ref[...] = xw[:, h:]
    s_ref[...] = jnp.dot(xw, as_ref[...], preferred_element_type=jnp.float32)
    d_ref[...] = jnp.dot(xw, ad_ref[...], preferred_element_type=jnp.float32)


def _project(x_pad, W, att_src, att_dst):
    NP, D = x_pad.shape
    H = D // 2
    RB = NP // 8
    return pl.pallas_call(
        _proj_body,
        grid=(NP // RB,),
        in_specs=[
            pl.BlockSpec((RB, D), lambda i: (i, 0)),
            pl.BlockSpec((D, D), lambda i: (0, 0)),
            pl.BlockSpec((D, 1), lambda i: (0, 0)),
            pl.BlockSpec((D, 1), lambda i: (0, 0)),
        ],
        out_specs=[
            pl.BlockSpec((RB, H), lambda i: (i, 0)),
            pl.BlockSpec((RB, H), lambda i: (i, 0)),
            pl.BlockSpec((RB, 1), lambda i: (i, 0)),
            pl.BlockSpec((RB, 1), lambda i: (i, 0)),
        ],
        out_shape=[
            jax.ShapeDtypeStruct((NP, H), jnp.float32),
            jax.ShapeDtypeStruct((NP, H), jnp.float32),
            jax.ShapeDtypeStruct((NP, 1), jnp.float32),
            jax.ShapeDtypeStruct((NP, 1), jnp.float32),
        ],
    )(x_pad, W, att_src.reshape(D, 1), att_dst.reshape(D, 1))


def _edge_ex(asrc, adst, src2, dst2, NP, EW, NB):
    RT = NP // NS

    mesh = plsc.VectorSubcoreMesh(core_axis_name="c", subcore_axis_name="s")

    @functools.partial(
        pl.kernel,
        mesh=mesh,
        out_type=[
            jax.ShapeDtypeStruct((NC * NP,), jnp.float32),
            jax.ShapeDtypeStruct((NW, 1, EW), jnp.float32),
        ],
        scratch_types=[
            pltpu.VMEM((NP,), jnp.float32),
            pltpu.VMEM((NP,), jnp.float32),
            pltpu.VMEM((NB, 128), jnp.int32),
            pltpu.VMEM((NB, 128), jnp.int32),
            pltpu.VMEM((EW,), jnp.float32),
            pltpu.SemaphoreType.DMA,
            pltpu.VMEM_SHARED((NP,), jnp.float32),
        ],
        compiler_params=pltpu.CompilerParams(needs_layout_passes=False),
    )
    def k(asrc_hbm, adst_hbm, src2_hbm, dst2_hbm,
          denomp_hbm, ex_hbm,
          asrc_v, adst_v, src2_v, dst2_v, exf_v, dsem, denom_sh):
        cid = lax.axis_index("c")
        sid = lax.axis_index("s")
        wid = sid * NC + cid

        pltpu.sync_copy(asrc_hbm, asrc_v)
        pltpu.sync_copy(adst_hbm, adst_v)
        pltpu.sync_copy(src2_hbm.at[wid], src2_v)
        pltpu.sync_copy(dst2_hbm.at[wid], dst2_v)
        zv = jnp.zeros((L,), jnp.float32)

        def zbody(i, carry):
            exf_v[pl.ds(i * L, L)] = zv
            return carry

        lax.fori_loop(0, (RT + L - 1) // L, zbody, 0)
        pltpu.sync_copy(exf_v.at[pl.ds(0, RT)],
                        denom_sh.at[pl.ds(sid * RT, RT)])
        plsc.subcore_barrier()

        def maxs_body(i, m):
            return jnp.maximum(m, asrc_v[pl.ds(i * L, L)])

        def maxd_body(i, m):
            return jnp.maximum(m, adst_v[pl.ds(i * L, L)])

        ms = lax.fori_loop(1, NP // L, maxs_body, asrc_v[pl.ds(0, L)])
        md = lax.fori_loop(1, NP // L, maxd_body, adst_v[pl.ds(0, L)])
        smax = jnp.max(ms) + jnp.max(md)
        cs = jnp.where(smax >= 0.0, smax, smax * 0.2)
        cv = jnp.full((L,), cs, jnp.float32)

        LAG = 8

        def ex_block(j):
            for kk in range(128 // L):
                off = j * 128 + kk * L
                sv = src2_v[j, pl.ds(kk * L, L)]
                dv = dst2_v[j, pl.ds(kk * L, L)]
                s = (plsc.load_gather(asrc_v, [sv])
                     + plsc.load_gather(adst_v, [dv]))
                alpha = jnp.where(s >= 0.0, s, s * 0.2)
                exf_v[pl.ds(off, L)] = jnp.exp(alpha - cv)
            pltpu.async_copy(exf_v.at[pl.ds(j * 128, 128)],
                             denom_sh.at[dst2_v.at[j]], dsem, add=True)

        def drain(j):
            pltpu.make_async_copy(exf_v.at[pl.ds(j * 128, 128)],
                                  denom_sh.at[dst2_v.at[j]], dsem).wait()

        def head_body(j, carry):
            ex_block(j)
            return carry

        def main_body(j, carry):
            ex_block(j)
            drain(j - LAG)
            return carry

        def tail_body(j, carry):
            drain(j)
            return carry

        lax.fori_loop(0, min(LAG, NB), head_body, 0)
        lax.fori_loop(min(LAG, NB), NB, main_body, 0)
        lax.fori_loop(max(0, NB - LAG), NB, tail_body, 0)
        pltpu.sync_copy(exf_v, ex_hbm.at[wid, 0])
        plsc.subcore_barrier()
        pltpu.sync_copy(denom_sh.at[pl.ds(sid * RT, RT)],
                        exf_v.at[pl.ds(0, RT)])
        pltpu.sync_copy(exf_v.at[pl.ds(0, RT)],
                        denomp_hbm.at[pl.ds(cid * NP + sid * RT, RT)])

    return k(asrc, adst, src2, dst2)


def _edge_msg(xw0, xw1, src2, dst2, ex, denomp, NP, EW2, NB2, D):
    RT = NP // NS
    H = D // 2

    mesh = plsc.VectorSubcoreMesh(core_axis_name="c", subcore_axis_name="s")

    @functools.partial(
        pl.kernel,
        mesh=mesh,
        out_type=jax.ShapeDtypeStruct((NC, NP, H), jnp.float32),
        scratch_types=[
            pltpu.VMEM((NP,), jnp.float32),
            pltpu.VMEM((EW2,), jnp.float32),
            pltpu.VMEM((3, 128), jnp.int32),
            pltpu.VMEM((NB2, 128), jnp.int32),
            pltpu.VMEM((128, H), jnp.float32),
            pltpu.VMEM((128, H), jnp.float32),
            pltpu.VMEM((128, H), jnp.float32),
            pltpu.SemaphoreType.DMA,
            pltpu.SemaphoreType.DMA,
            pltpu.SemaphoreType.DMA,
            pltpu.SemaphoreType.DMA,
            pltpu.SemaphoreType.DMA,
            pltpu.SemaphoreType.DMA,
            pltpu.SemaphoreType.DMA,
            pltpu.SemaphoreType.DMA,
            pltpu.SemaphoreType.DMA,
            pltpu.VMEM_SHARED((NP, H), jnp.float32),
        ],
        compiler_params=pltpu.CompilerParams(needs_layout_passes=False,
                                             use_tc_tiling_on_sc=False),
    )
    def k(xw0_hbm, xw1_hbm, src2_hbm, dst2_hbm, ex_hbm, denomp_hbm,
          outp_hbm,
          den0_v, coef_v, srcring_v, dst2_v, rows_v, rows2_v, rows3_v,
          sem, sem2, sem3, isem, isem2, isem3, ssem, ssem2, ssem3, out_sh):
        cid = lax.axis_index("c")
        sid = lax.axis_index("s")

        pltpu.sync_copy(denomp_hbm.at[pl.ds(0, NP)], den0_v)
        pltpu.sync_copy(denomp_hbm.at[pl.ds(NP, NP)], coef_v.at[pl.ds(0, NP)])
        pltpu.sync_copy(dst2_hbm.at[sid], dst2_v)
        zv = jnp.zeros((L,), jnp.float32)

        def zbody(r, carry):
            for kk in range(H // L):
                rows_v[r, pl.ds(kk * L, L)] = zv
            return carry

        lax.fori_loop(0, 128, zbody, 0)
        nfull = RT // 128
        rem = RT % 128
        for q in range(nfull):
            pltpu.sync_copy(rows_v,
                            out_sh.at[pl.ds(sid * RT + q * 128, 128)])
        if rem:
            pltpu.sync_copy(rows_v.at[pl.ds(0, rem)],
                            out_sh.at[pl.ds(sid * RT + nfull * 128, rem)])

        def dbody(i, carry):
            for u in range(8):
                sl = pl.ds((i * 8 + u) * L, L)
                den0_v[sl] = 1.0 / (den0_v[sl] + coef_v[sl] + 1e-16)
            return carry

        lax.fori_loop(0, NP // L // 8, dbody, 0)
        pltpu.sync_copy(ex_hbm.at[sid, 0], coef_v)

        def cbody(j, carry):
            for kk in range(128 // L):
                off = j * 128 + kk * L
                dv = dst2_v[j, pl.ds(kk * L, L)]
                r = plsc.load_gather(den0_v, [dv])
                coef_v[pl.ds(off, L)] = coef_v[pl.ds(off, L)] * r
            return carry

        lax.fori_loop(0, NB2, cbody, 0)
        plsc.subcore_barrier()

        def scale(buf, j):
            def gbody(g, c2):
                base = j * 128 + g * L
                for r16 in range(L):
                    r = g * L + r16
                    cb = plsc.load_gather(
                        coef_v, [jnp.full((L,), base + r16, jnp.int32)])
                    for kk in range(H // L):
                        sl = pl.ds(kk * L, L)
                        buf[r, sl] = buf[r, sl] * cb
                return c2

            lax.fori_loop(0, 128 // L, gbody, 0)

        bufs = (rows_v, rows2_v, rows3_v)
        sems = (sem, sem2, sem3)
        isems = (isem, isem2, isem3)
        ssems = (ssem, ssem2, ssem3)

        def run(xwh_hbm):
            for b in range(min(3, NB2)):
                pltpu.async_copy(src2_hbm.at[sid, b], srcring_v.at[b],
                                 isems[b])
            for b in range(min(2, NB2)):
                pltpu.make_async_copy(src2_hbm.at[sid, b], srcring_v.at[b],
                                      isems[b]).wait()
                pltpu.async_copy(xwh_hbm.at[srcring_v.at[b]], bufs[b],
                                 sems[b])

            def pbody(t, carry):
                for ph in range(3):
                    j = 3 * t + ph
                    pltpu.make_async_copy(
                        xwh_hbm.at[srcring_v.at[ph]], bufs[ph],
                        sems[ph]).wait()

                    @pl.when(j + 3 < NB2)
                    def _(ph=ph, j=j):
                        pltpu.async_copy(src2_hbm.at[sid, j + 3],
                                         srcring_v.at[ph], isems[ph])

                    scale(bufs[ph], j)
                    pltpu.async_copy(bufs[ph], out_sh.at[dst2_v.at[j]],
                                     ssems[ph], add=True)

                    pn = (ph + 2) % 3

                    @pl.when((j + 2 < NB2) & (j >= 1))
                    def _(j=j, pn=pn):
                        pltpu.make_async_copy(
                            bufs[pn], out_sh.at[dst2_v.at[j - 1]],
                            ssems[pn]).wait()

                    @pl.when(j + 2 < NB2)
                    def _(ph=ph, j=j, pn=pn):
                        pltpu.make_async_copy(
                            src2_hbm.at[sid, j + 2], srcring_v.at[pn],
                            isems[pn]).wait()
                        pltpu.async_copy(
                            xwh_hbm.at[srcring_v.at[pn]], bufs[pn],
                            sems[pn])
                return carry

            lax.fori_loop(0, NB2 // 3, pbody, 0)
            nw = 3 * (NB2 // 3)
            for j in range(max(0, nw - 3), nw):
                if not (j + 3 < NB2 and j + 1 >= 1):
                    b = j % 3
                    pltpu.make_async_copy(
                        bufs[b], out_sh.at[dst2_v.at[j]], ssems[b]).wait()
            base = (NB2 // 3) * 3
            for q in range(NB2 % 3):
                j = base + q
                b = j % 3
                pltpu.make_async_copy(
                    xwh_hbm.at[srcring_v.at[b]], bufs[b], sems[b]).wait()
                scale(bufs[b], j)
                pltpu.sync_copy(bufs[b], out_sh.at[dst2_v.at[j]], add=True)

        @pl.when(cid == 0)
        def _():
            run(xw0_hbm)

        @pl.when(cid == 1)
        def _():
            run(xw1_hbm)

        plsc.subcore_barrier()
        for q in range(nfull):
            base = sid * RT + q * 128
            pltpu.sync_copy(out_sh.at[pl.ds(base, 128)], rows_v)
            pltpu.sync_copy(rows_v, outp_hbm.at[cid, pl.ds(base, 128)])
        if rem:
            base = sid * RT + nfull * 128
            pltpu.sync_copy(out_sh.at[pl.ds(base, rem)],
                            rows_v.at[pl.ds(0, rem)])
            pltpu.sync_copy(rows_v.at[pl.ds(0, rem)],
                            outp_hbm.at[cid, pl.ds(base, rem)])

    return k(xw0, xw1, src2, dst2, ex, denomp)


def _fin_body(p_ref, x_ref, b_ref, o_ref):
    p = p_ref[...]
    h = jnp.concatenate([p[0], p[1]], axis=1) + b_ref[...]
    o_ref[...] = x_ref[...] + jnp.where(h > 0.0, h, jnp.exp(h) - 1.0)


def _finalize(outp, x, bias, R):
    N, D = x.shape
    H = D // 2
    return pl.pallas_call(
        _fin_body,
        grid=(N // R,),
        in_specs=[
            pl.BlockSpec((2, R, H), lambda i: (0, i, 0)),
            pl.BlockSpec((R, D), lambda i: (i, 0)),
            pl.BlockSpec((1, D), lambda i: (0, 0)),
        ],
        out_specs=pl.BlockSpec((R, D), lambda i: (i, 0)),
        out_shape=jax.ShapeDtypeStruct((N, D), jnp.float32),
    )(outp, x, bias.reshape(1, D))


def kernel(x, edge_index, W, att_src, att_dst, bias):
    N, D = x.shape
    E = edge_index.shape[1]
    NP = ((N + 1 + 127) // 128) * 128
    ET0 = E + N
    NB = -(-ET0 // (NW * 128))
    EW = NB * 128
    ET = NW * EW

    loop = jnp.arange(N, dtype=jnp.int32)
    pad = ET - ET0
    src_all = jnp.concatenate(
        [edge_index[0], loop, jnp.zeros((pad,), jnp.int32)])
    dst_all = jnp.concatenate(
        [edge_index[1], loop, jnp.full((pad,), N, jnp.int32)])
    src2 = src_all.reshape(NW, NB, 128)
    dst2 = dst_all.reshape(NW, NB, 128)

    x_pad = jnp.zeros((NP, D), jnp.float32).at[:N].set(x)
    xw0, xw1, as_col, ad_col = _project(x_pad, W, att_src, att_dst)
    asrc = as_col.reshape(NP)
    adst = ad_col.reshape(NP)

    denomp, ex = _edge_ex(asrc, adst, src2, dst2, NP, EW, NB)

    EW2 = NC * EW
    NB2 = NC * NB
    srcb2 = src_all.reshape(NS, NB2, 128)
    dstb2 = dst_all.reshape(NS, NB2, 128)
    exb = ex.reshape(NS, 1, EW2)
    outp = _edge_msg(xw0, xw1, srcb2, dstb2, exb, denomp, NP, EW2, NB2, D)

    R = next((r for r in range(512, 7, -8) if N % r == 0), None)
    if R is None:
        R = next(r for r in (7, 5, 4, 2, 1) if N % r == 0)
    return _finalize(outp, x, bias, R)
